# baseline (speedup 1.0000x reference)
import jax
import jax.numpy as jnp
from jax.experimental import pallas as pl
from jax.experimental.pallas import tpu as pltpu


def _fused_row_block(adj_ref, x_ref, h0_ref, xb_ref, w1_ref, w2_ref, o_ref):
    hi = jnp.dot(adj_ref[...], x_ref[...], preferred_element_type=jnp.float32)
    o_ref[...] = (
        jnp.dot(hi, w1_ref[...], preferred_element_type=jnp.float32)
        + jnp.dot(h0_ref[...], w2_ref[...], preferred_element_type=jnp.float32)
        + xb_ref[...]
    )


def kernel(input, adj, h0, lamda, alpha, l, weight):
    n, d = input.shape
    theta = jnp.minimum(1.0, jnp.log(lamda / l + 1.0))
    wp = theta * weight + (1.0 - theta) * jnp.eye(d, dtype=weight.dtype)
    w1 = (1.0 - alpha) * wp
    w2 = alpha * wp

    bm = 400 if n % 400 == 0 else n

    return pl.pallas_call(
        _fused_row_block,
        grid=(n // bm,),
        in_specs=[
            pl.BlockSpec((bm, n), lambda i: (i, 0)),
            pl.BlockSpec((n, d), lambda i: (0, 0)),
            pl.BlockSpec((bm, d), lambda i: (i, 0)),
            pl.BlockSpec((bm, d), lambda i: (i, 0)),
            pl.BlockSpec((d, d), lambda i: (0, 0)),
            pl.BlockSpec((d, d), lambda i: (0, 0)),
        ],
        out_specs=pl.BlockSpec((bm, d), lambda i: (i, 0)),
        out_shape=jax.ShapeDtypeStruct((n, d), jnp.float32),
        compiler_params=pltpu.CompilerParams(
            dimension_semantics=("arbitrary",),
        ),
    )(adj, input, h0, input, w1, w2)


# speedup vs baseline: 1.0022x; 1.0022x over previous
import jax
import jax.numpy as jnp
from jax.experimental import pallas as pl
from jax.experimental.pallas import tpu as pltpu


def _fused_row_block(adj_ref, x_ref, h0_ref, w1_ref, w2_ref, o_ref, *, bm):
    i = pl.program_id(0)
    hi = jnp.dot(adj_ref[...], x_ref[...], preferred_element_type=jnp.float32)
    xb = x_ref[pl.ds(i * bm, bm), :]
    o_ref[...] = (
        jnp.dot(hi, w1_ref[...], preferred_element_type=jnp.float32)
        + jnp.dot(h0_ref[...], w2_ref[...], preferred_element_type=jnp.float32)
        + xb
    )


def kernel(input, adj, h0, lamda, alpha, l, weight):
    n, d = input.shape
    theta = jnp.minimum(1.0, jnp.log(lamda / l + 1.0))
    wp = theta * weight + (1.0 - theta) * jnp.eye(d, dtype=weight.dtype)
    w1 = (1.0 - alpha) * wp
    w2 = alpha * wp

    bm = 200 if n % 200 == 0 else n

    import functools
    body = functools.partial(_fused_row_block, bm=bm)
    return pl.pallas_call(
        body,
        grid=(n // bm,),
        in_specs=[
            pl.BlockSpec((bm, n), lambda i: (i, 0)),
            pl.BlockSpec((n, d), lambda i: (0, 0)),
            pl.BlockSpec((bm, d), lambda i: (i, 0)),
            pl.BlockSpec((d, d), lambda i: (0, 0)),
            pl.BlockSpec((d, d), lambda i: (0, 0)),
        ],
        out_specs=pl.BlockSpec((bm, d), lambda i: (i, 0)),
        out_shape=jax.ShapeDtypeStruct((n, d), jnp.float32),
        compiler_params=pltpu.CompilerParams(
            dimension_semantics=("arbitrary",),
        ),
    )(adj, input, h0, w1, w2)
